# baseline (speedup 1.0000x reference)
import jax
import jax.numpy as jnp
from jax import lax
from jax.experimental import pallas as pl
from jax.experimental.pallas import tpu as pltpu
from jax.experimental.pallas import tpu_sc as plsc

B = 4096
D = 32
W = 10
NNEG = 50
NPAIR = W + NNEG

NC, NS = 2, 16
NW = NC * NS
BPW = B // NW
G = 16
NGROUP = BPW // G


def _sc_scores_body(tgt_hbm, ctx_hbm, neg_hbm, emb_in_hbm, emb_out_hbm,
                    scores_hbm,
                    tgt_i0, tgt_i1, ctx_i0, ctx_i1, neg_i0, neg_i1,
                    tgt_r0, tgt_r1, ctx_r0, ctx_r1, neg_r0, neg_r1,
                    score_v, sem_i0, sem_i1, sem_g0, sem_g1):
    wid = lax.axis_index("s") * NC + lax.axis_index("c")
    base = wid * BPW

    tgt_i = (tgt_i0, tgt_i1)
    ctx_i = (ctx_i0, ctx_i1)
    neg_i = (neg_i0, neg_i1)
    tgt_r = (tgt_r0, tgt_r1)
    ctx_r = (ctx_r0, ctx_r1)
    neg_r = (neg_r0, neg_r1)
    sem_i = (sem_i0, sem_i1)
    sem_g = (sem_g0, sem_g1)

    def idx_copies(g, p):
        b0 = base + g * G
        return [
            (tgt_hbm.at[pl.ds(b0, G)], tgt_i[p]),
            (ctx_hbm.at[pl.ds(b0 * W, G * W)], ctx_i[p]),
            (neg_hbm.at[pl.ds(b0 * NNEG, G * NNEG)], neg_i[p]),
        ]

    def gather_copies(p):
        out = [(emb_in_hbm.at[tgt_i[p]], tgt_r[p])]
        for off in range(0, G * W, 128):
            sz = min(128, G * W - off)
            out.append((emb_out_hbm.at[ctx_i[p].at[pl.ds(off, sz)]],
                        ctx_r[p].at[pl.ds(off, sz), :]))
        for off in range(0, G * NNEG, 128):
            sz = min(128, G * NNEG - off)
            out.append((emb_out_hbm.at[neg_i[p].at[pl.ds(off, sz)]],
                        neg_r[p].at[pl.ds(off, sz), :]))
        return out

    def fire(pairs, sem):
        for s, d in pairs:
            pltpu.async_copy(s, d, sem)

    def drain(pairs, sem):
        for s, d in pairs:
            pltpu.make_async_copy(s, d, sem).wait()

    lane = lax.iota(jnp.int32, 16)

    def compute(g, p):
        chunks = [(ctx_r[p], lane * W, 0, False)]
        for k in range(NNEG // W):
            chunks.append((neg_r[p], lane * NNEG + k * W, W + k * W, True))
        for rows_ref, rowbase, j0, negate in chunks:
            def body(d, accs, rows_ref=rows_ref, rowbase=rowbase, p=p):
                col = jnp.full((16,), d, jnp.int32)
                tv = plsc.load_gather(tgt_r[p], [lane, col])
                return tuple(
                    accs[j] + tv * plsc.load_gather(rows_ref,
                                                    [rowbase + j, col])
                    for j in range(W))
            accs = lax.fori_loop(
                0, D, body,
                tuple(jnp.zeros((16,), jnp.float32) for _ in range(W)))
            for j in range(W):
                val = -accs[j] if negate else accs[j]
                score_v[j0 + j, pl.ds(g * G, G)] = val

    fire(idx_copies(0, 0), sem_i[0])
    fire(idx_copies(1, 1), sem_i[1])
    drain(idx_copies(0, 0), sem_i[0])
    fire(gather_copies(0), sem_g[0])
    for g in range(NGROUP):
        p = g & 1
        if g + 1 < NGROUP:
            drain(idx_copies(g + 1, 1 - p), sem_i[1 - p])
            fire(gather_copies(1 - p), sem_g[1 - p])
        drain(gather_copies(p), sem_g[p])
        if g + 2 < NGROUP:
            fire(idx_copies(g + 2, p), sem_i[p])
        compute(g, p)
    pltpu.sync_copy(score_v, scores_hbm.at[:, pl.ds(base, BPW)])


_sc_scores = pl.kernel(
    out_type=jax.ShapeDtypeStruct((NPAIR, B), jnp.float32),
    mesh=plsc.VectorSubcoreMesh(core_axis_name="c", subcore_axis_name="s"),
    compiler_params=pltpu.CompilerParams(needs_layout_passes=False,
                                         use_tc_tiling_on_sc=False),
    scratch_types=[
        pltpu.VMEM((G,), jnp.int32), pltpu.VMEM((G,), jnp.int32),
        pltpu.VMEM((G * W,), jnp.int32), pltpu.VMEM((G * W,), jnp.int32),
        pltpu.VMEM((G * NNEG,), jnp.int32), pltpu.VMEM((G * NNEG,), jnp.int32),
        pltpu.VMEM((G, D), jnp.float32), pltpu.VMEM((G, D), jnp.float32),
        pltpu.VMEM((G * W, D), jnp.float32), pltpu.VMEM((G * W, D), jnp.float32),
        pltpu.VMEM((G * NNEG, D), jnp.float32),
        pltpu.VMEM((G * NNEG, D), jnp.float32),
        pltpu.VMEM((NPAIR, BPW), jnp.float32),
        pltpu.SemaphoreType.DMA, pltpu.SemaphoreType.DMA,
        pltpu.SemaphoreType.DMA, pltpu.SemaphoreType.DMA,
    ],
)(_sc_scores_body)


def _tc_loss_body(s_ref, o_ref):
    x = s_ref[...]
    o_ref[0, 0] = -jnp.sum(jax.nn.log_sigmoid(x)) / (B * W)


_tc_loss = pl.pallas_call(
    _tc_loss_body,
    out_shape=jax.ShapeDtypeStruct((1, 1), jnp.float32),
    out_specs=pl.BlockSpec(memory_space=pltpu.SMEM),
)


def kernel(target, context, negative_samples, emb_in, emb_out):
    tgt_idx = target.reshape(-1).astype(jnp.int32)
    ctx_idx = context.reshape(-1).astype(jnp.int32)
    neg_idx = negative_samples.reshape(-1).astype(jnp.int32)
    scores = _sc_scores(tgt_idx, ctx_idx, neg_idx, emb_in, emb_out)
    return _tc_loss(scores)[0, 0]
